# Initial kernel scaffold; baseline (speedup 1.0000x reference)
import jax
import jax.numpy as jnp
from jax.experimental import pallas as pl


def kernel(x, edge_index, x_label, W_label, W1, b1, W2, b2):
    raise NotImplementedError("write your pallas kernel here")


# speedup vs baseline: 1.3357x; 1.3357x over previous
import functools

import jax
import jax.numpy as jnp
from jax import lax
from jax.experimental import pallas as pl
from jax.experimental.pallas import tpu as pltpu
from jax.experimental.pallas import tpu_sc as plsc

N, E, F = 10000, 320000, 128
NC, NS = 2, 16
NW = NC * NS
L = 16
NSL = F // L

EW = E // NW
CB = 80
NCH = EW // CB

PT = 320
NLAST = N - (NW - 1) * PT
DC = 2000
MB_CAP = 128
FLUSH_AT = MB_CAP - L

BN = 1000
BE = 2560


def _uv_body(x_ref, w_ref, b_ref, u_ref, v_ref):
    uv = jnp.dot(x_ref[...], w_ref[...], preferred_element_type=jnp.float32)
    uv = uv + b_ref[...]
    u_ref[...] = uv[:, :F]
    v_ref[...] = uv[:, F:]


def _mlp_body(xl_ref, g_ref, w1t_ref, w2t_ref, b2_ref, m_ref):
    l1 = jnp.dot(xl_ref[...], w1t_ref[...], preferred_element_type=jnp.float32)
    h = jnp.maximum(g_ref[...] + l1, 0.0)
    m_ref[...] = (
        jnp.dot(h, w2t_ref[...], preferred_element_type=jnp.float32) + b2_ref[...]
    )


def _gather_body(u_hbm, v_hbm, dst_hbm, src_hbm, g_hbm,
                 didx, sidx, urows, vrows, usem, vsem):
    wid = lax.axis_index("s") * NC + lax.axis_index("c")
    base = wid * EW
    pltpu.sync_copy(dst_hbm.at[pl.ds(base, EW)], didx)
    pltpu.sync_copy(src_hbm.at[pl.ds(base, EW)], sidx)

    def chunk(c, carry):
        off = c * CB
        cu = pltpu.async_copy(u_hbm.at[didx.at[pl.ds(off, CB)]], urows, usem)
        cv = pltpu.async_copy(v_hbm.at[sidx.at[pl.ds(off, CB)]], vrows, vsem)
        cu.wait()
        cv.wait()

        def addrow(r, carry2):
            for j in range(NSL):
                sl = pl.ds(j * L, L)
                plsc.addupdate(urows.at[r, sl], vrows[r, sl])
            return carry2

        lax.fori_loop(0, CB, addrow, 0)
        pltpu.sync_copy(urows, g_hbm.at[pl.ds(base + off, CB)])
        return carry

    lax.fori_loop(0, NCH, chunk, 0)


def _segmax_body(dst_hbm, m_hbm, out_hbm, dstc, mid, mld, rows, acc, gsem):
    wid = lax.axis_index("s") * NC + lax.axis_index("c")
    lo = wid * PT
    neg_inf = jnp.full((L,), -jnp.inf, dtype=jnp.float32)

    def init_acc(i, carry):
        for j in range(NSL):
            acc[i, pl.ds(j * L, L)] = neg_inf
        return carry

    lax.fori_loop(0, PT, init_acc, 0)
    zeros_i = jnp.zeros((L,), dtype=jnp.int32)
    for b in range(MB_CAP // L):
        mid[pl.ds(b * L, L)] = zeros_i
    for b in range(MB_CAP // L + 1):
        mld[pl.ds(b * L, L)] = zeros_i

    def flush(nm):
        pltpu.async_copy(m_hbm.at[mid], rows, gsem).wait()

        def apply(k, carry):
            ld = mld[pl.ds(k, L)][0]
            for j in range(NSL):
                sl = pl.ds(j * L, L)
                acc[ld, sl] = jnp.maximum(acc[ld, sl], rows[k, sl])
            return carry

        lax.fori_loop(0, nm, apply, 0)
        return jnp.int32(0)

    iota = lax.iota(jnp.int32, L)

    def chunk(c, nm):
        pltpu.sync_copy(dst_hbm.at[pl.ds(c * DC, DC)], dstc)

        def step(s, nm):
            d = dstc[pl.ds(s * L, L)]
            ldv = d - lo
            msk = (ldv >= 0) & (ldv < PT)
            cnt = jnp.sum(msk.astype(jnp.int32))
            eids = (c * DC + s * L) + iota
            plsc.store_compressed(mid.at[pl.ds(nm, L)], eids, mask=msk)
            plsc.store_compressed(mld.at[pl.ds(nm, L)], ldv, mask=msk)
            nm = nm + cnt
            return lax.cond(nm >= FLUSH_AT, flush, lambda x: x, nm)

        return lax.fori_loop(0, DC // L, step, nm)

    nm = lax.fori_loop(0, E // DC, chunk, jnp.int32(0))
    flush(nm)

    def finish(i, carry):
        for j in range(NSL):
            sl = pl.ds(j * L, L)
            v = acc[i, sl]
            acc[i, sl] = jnp.where(v == -jnp.inf, 0.0, v)
        return carry

    lax.fori_loop(0, PT, finish, 0)

    @pl.when(wid < NW - 1)
    def _():
        pltpu.sync_copy(acc, out_hbm.at[pl.ds(lo, PT)])

    @pl.when(wid == NW - 1)
    def _():
        pltpu.sync_copy(acc.at[pl.ds(0, NLAST)], out_hbm.at[pl.ds(lo, NLAST)])


_sc_mesh = plsc.VectorSubcoreMesh(
    core_axis_name="c", subcore_axis_name="s", num_cores=NC, num_subcores=NS
)

_sc_params = pltpu.CompilerParams(needs_layout_passes=False)

_gather_kernel = functools.partial(
    pl.kernel,
    mesh=_sc_mesh,
    compiler_params=_sc_params,
    out_type=jax.ShapeDtypeStruct((E, F), jnp.float32),
    scratch_types=[
        pltpu.VMEM((EW,), jnp.int32),
        pltpu.VMEM((EW,), jnp.int32),
        pltpu.VMEM((CB, F), jnp.float32),
        pltpu.VMEM((CB, F), jnp.float32),
        pltpu.SemaphoreType.DMA,
        pltpu.SemaphoreType.DMA,
    ],
)(_gather_body)

_segmax_kernel = functools.partial(
    pl.kernel,
    mesh=_sc_mesh,
    compiler_params=_sc_params,
    out_type=jax.ShapeDtypeStruct((N, F), jnp.float32),
    scratch_types=[
        pltpu.VMEM((DC,), jnp.int32),
        pltpu.VMEM((MB_CAP,), jnp.int32),
        pltpu.VMEM((MB_CAP + L,), jnp.int32),
        pltpu.VMEM((MB_CAP, F), jnp.float32),
        pltpu.VMEM((PT, F), jnp.float32),
        pltpu.SemaphoreType.DMA,
    ],
)(_segmax_body)


def kernel(x, edge_index, x_label, W_label, W1, b1, W2, b2):
    src = edge_index[0]
    dst = edge_index[1]
    P = W_label[:, :F].T
    Q = W_label[:, F:].T
    w_uv = jnp.concatenate([(P - Q) @ W1.T, Q @ W1.T], axis=1)
    b_uv = jnp.concatenate([b1, jnp.zeros((F,), jnp.float32)])[None, :]

    u, v = pl.pallas_call(
        _uv_body,
        grid=(N // BN,),
        in_specs=[
            pl.BlockSpec((BN, F), lambda i: (i, 0)),
            pl.BlockSpec((F, 2 * F), lambda i: (0, 0)),
            pl.BlockSpec((1, 2 * F), lambda i: (0, 0)),
        ],
        out_specs=[
            pl.BlockSpec((BN, F), lambda i: (i, 0)),
            pl.BlockSpec((BN, F), lambda i: (i, 0)),
        ],
        out_shape=[
            jax.ShapeDtypeStruct((N, F), jnp.float32),
            jax.ShapeDtypeStruct((N, F), jnp.float32),
        ],
    )(x, w_uv, b_uv)

    g = _gather_kernel(u, v, dst, src)

    m = pl.pallas_call(
        _mlp_body,
        grid=(E // BE,),
        in_specs=[
            pl.BlockSpec((BE, F), lambda i: (i, 0)),
            pl.BlockSpec((BE, F), lambda i: (i, 0)),
            pl.BlockSpec((F, F), lambda i: (0, 0)),
            pl.BlockSpec((F, F), lambda i: (0, 0)),
            pl.BlockSpec((1, F), lambda i: (0, 0)),
        ],
        out_specs=pl.BlockSpec((BE, F), lambda i: (i, 0)),
        out_shape=jax.ShapeDtypeStruct((E, F), jnp.float32),
    )(x_label, g, W1.T, W2.T, b2[None, :])

    return _segmax_kernel(dst, m)
